# baseline (speedup 1.0000x reference)
import jax
import jax.numpy as jnp
from jax import lax
from jax.experimental import pallas as pl
from jax.experimental.pallas import tpu as pltpu
from jax.experimental.pallas import tpu_sc as plsc

_N = 1000000
_D = 16
_CHUNK = 62464
_TAIL_OFF = 16 * _CHUNK
_TAIL_BLOCKS = 5


def _sc_body(u_in, i_in, u_out, i_out):
    c = lax.axis_index("c")
    s = lax.axis_index("s")
    w = s * 2 + c
    g = pl.multiple_of(8 * (w // 16), 8)
    off = pl.multiple_of((w % 16) * _CHUNK, 128)
    rows = pl.ds(g, 8)
    cols = pl.ds(off, _CHUNK)
    pltpu.sync_copy(u_in.at[rows, cols], u_out.at[rows, cols])
    pltpu.sync_copy(i_in.at[rows, cols], i_out.at[rows, cols])


def _tail_body(uo_b, io_b, u_b, i_b, uo_o, io_o):
    del uo_b, io_b
    uo_o[...] = u_b[...]
    io_o[...] = i_b[...]


def kernel(adj, user_emb, item_emb):
    del adj
    ut = user_emb.T
    it = item_emb.T
    mesh = plsc.VectorSubcoreMesh(core_axis_name="c", subcore_axis_name="s")
    uo, io = pl.kernel(
        _sc_body,
        out_type=(
            jax.ShapeDtypeStruct((_D, _N), jnp.float32),
            jax.ShapeDtypeStruct((_D, _N), jnp.float32),
        ),
        mesh=mesh,
    )(ut, it)

    tail_spec = pl.BlockSpec((_D, 128), lambda g: (0, _TAIL_OFF // 128 + g))
    uo, io = pl.pallas_call(
        _tail_body,
        grid=(_TAIL_BLOCKS,),
        in_specs=[tail_spec, tail_spec, tail_spec, tail_spec],
        out_specs=(tail_spec, tail_spec),
        out_shape=(
            jax.ShapeDtypeStruct((_D, _N), jnp.float32),
            jax.ShapeDtypeStruct((_D, _N), jnp.float32),
        ),
        input_output_aliases={0: 0, 1: 1},
    )(uo, io, ut, it)
    return uo.T, io.T


# speedup vs baseline: 34.5832x; 34.5832x over previous
import jax
import jax.numpy as jnp
from jax import lax
from jax.experimental import pallas as pl
from jax.experimental.pallas import tpu as pltpu
from jax.experimental.pallas import tpu_sc as plsc

_N = 1000000
_D = 16
_CHUNK = 62464
_TAIL_OFF = 16 * _CHUNK
_TAIL_BLOCKS = 5


_W = 7808
_NSUB = _CHUNK // _W


def _sc_body(u_in, i_in, u_out, i_out, b0, b1, si0, si1, so0, so1):
    c = lax.axis_index("c")
    s = lax.axis_index("s")
    w = s * 2 + c
    g = pl.multiple_of(8 * (w // 16), 8)
    off = pl.multiple_of((w % 16) * _CHUNK, 128)
    rows = pl.ds(g, 8)

    bufs = (b0, b1)
    isems = (si0, si1)
    osems = (so0, so1)

    out_cps = [None, None]
    for src, dst in ((u_in, u_out), (i_in, i_out)):
        for k in range(_NSUB):
            b = k % 2
            cols = pl.ds(pl.multiple_of(off + k * _W, 128), _W)
            if out_cps[b] is not None:
                out_cps[b].wait()
            in_cp = pltpu.make_async_copy(src.at[rows, cols], bufs[b],
                                          isems[b])
            in_cp.start()
            in_cp.wait()
            out_cp = pltpu.make_async_copy(bufs[b], dst.at[rows, cols],
                                           osems[b])
            out_cp.start()
            out_cps[b] = out_cp
    for b in range(2):
        if out_cps[b] is not None:
            out_cps[b].wait()


def _tail_body(uo_b, io_b, u_b, i_b, uo_o, io_o):
    del uo_b, io_b
    uo_o[...] = u_b[...]
    io_o[...] = i_b[...]


def kernel(adj, user_emb, item_emb):
    del adj
    ut = user_emb.T
    it = item_emb.T
    mesh = plsc.VectorSubcoreMesh(core_axis_name="c", subcore_axis_name="s")
    uo, io = pl.kernel(
        _sc_body,
        out_type=(
            jax.ShapeDtypeStruct((_D, _N), jnp.float32),
            jax.ShapeDtypeStruct((_D, _N), jnp.float32),
        ),
        mesh=mesh,
        scratch_types=[
            pltpu.VMEM((8, _W), jnp.float32),
            pltpu.VMEM((8, _W), jnp.float32),
            pltpu.SemaphoreType.DMA,
            pltpu.SemaphoreType.DMA,
            pltpu.SemaphoreType.DMA,
            pltpu.SemaphoreType.DMA,
        ],
    )(ut, it)

    tail_spec = pl.BlockSpec((_D, 128), lambda g: (0, _TAIL_OFF // 128 + g))
    uo, io = pl.pallas_call(
        _tail_body,
        grid=(_TAIL_BLOCKS,),
        in_specs=[tail_spec, tail_spec, tail_spec, tail_spec],
        out_specs=(tail_spec, tail_spec),
        out_shape=(
            jax.ShapeDtypeStruct((_D, _N), jnp.float32),
            jax.ShapeDtypeStruct((_D, _N), jnp.float32),
        ),
        input_output_aliases={0: 0, 1: 1},
    )(uo, io, ut, it)
    return uo.T, io.T


# speedup vs baseline: 49.3409x; 1.4267x over previous
import jax
import jax.numpy as jnp
from jax.experimental import pallas as pl
from jax.experimental.pallas import tpu as pltpu

_N = 1000000
_D = 16
_BLOCK = 98304
_GRID = (_N + _BLOCK - 1) // _BLOCK


def _copy_body(u_in, i_in, u_out, i_out):
    u_out[...] = u_in[...]
    i_out[...] = i_in[...]


def kernel(adj, user_emb, item_emb):
    del adj
    ut = user_emb.T
    it = item_emb.T
    spec = pl.BlockSpec((_D, _BLOCK), lambda g: (0, g))
    uo, io = pl.pallas_call(
        _copy_body,
        grid=(_GRID,),
        in_specs=[spec, spec],
        out_specs=(spec, spec),
        out_shape=(
            jax.ShapeDtypeStruct((_D, _N), jnp.float32),
            jax.ShapeDtypeStruct((_D, _N), jnp.float32),
        ),
    )(ut, it)
    return uo.T, io.T
